# baseline (speedup 1.0000x reference)
import functools

import jax
import jax.numpy as jnp
from jax.experimental import pallas as pl

_K = 20
_BETA = 1.0
_LAMBDA = 1.0
_EPS = 1e-06


def _knn_tile_kernel(q_ref, pt_ref, outf_ref, outi_ref, *, n, k):
    q = q_ref[0]
    pt = pt_ref[0]
    r = q.shape[0]

    qsq = jnp.sum(q * q, axis=1, keepdims=True)
    psq = jnp.sum(pt * pt, axis=0, keepdims=True)
    dot = jax.lax.dot(q, pt, preferred_element_type=jnp.float32)
    dist2 = jnp.maximum((qsq + psq) - 2.0 * dot, 0.0)

    inf = jnp.float32(jnp.inf)
    iota_col = jax.lax.broadcasted_iota(jnp.int32, (n, 2), 0).astype(jnp.float32)
    ones_col = jnp.ones((n, 2), dtype=jnp.float32)
    sel_col = jnp.where(
        jax.lax.broadcasted_iota(jnp.int32, (n, 2), 1) == 0, iota_col, ones_col)

    vals = dist2
    mins = []
    idxs = []
    for _ in range(k + 1):
        m = jnp.min(vals, axis=1, keepdims=True)
        eq = vals == m
        sc = jax.lax.dot(eq.astype(jnp.float32), sel_col,
                         precision=jax.lax.Precision.HIGHEST,
                         preferred_element_type=jnp.float32)
        s_idx = sc[:, 0:1]
        tie = jnp.any(sc[:, 1:2] > 1.5)

        def _slow(args):
            eq_, vals_ = args
            iotai = jax.lax.broadcasted_iota(jnp.int32, (r, n), 1)
            cand = jnp.where(eq_, iotai, jnp.int32(n))
            idx = jnp.min(cand, axis=1, keepdims=True)
            nv = jnp.where(iotai == idx, inf, vals_)
            return idx.astype(jnp.float32), nv

        def _fast(args):
            eq_, vals_ = args
            return s_idx, jnp.where(eq_, inf, vals_)

        idxf, vals = jax.lax.cond(tie, _slow, _fast, (eq, vals))
        mins.append(m)
        idxs.append(idxf)

    md = jnp.concatenate(mins, axis=1)
    mi = jnp.concatenate(idxs, axis=1).astype(jnp.int32)

    d = jnp.sqrt(jnp.maximum(md[:, 1:], 1e-12))
    d_i = d[:, k - 1:k]
    sigma = _BETA * d_i + _EPS
    s_local = _LAMBDA * d_i * d_i
    w = jnp.exp(-(d * d) / (2.0 * sigma * sigma))

    padf = jnp.zeros((r, 32 - k - 1), dtype=jnp.float32)
    outf_ref[0] = jnp.concatenate([w, s_local, padf], axis=1)
    padi = jnp.zeros((r, 32 - k), dtype=jnp.int32)
    outi_ref[0] = jnp.concatenate([mi[:, 1:], padi], axis=1)


def kernel(point_cloud):
    b, n, _ = point_cloud.shape
    k = _K
    row_tile = 256
    pc_t = jnp.transpose(point_cloud, (0, 2, 1))

    outf, outi = pl.pallas_call(
        functools.partial(_knn_tile_kernel, n=n, k=k),
        grid=(b, n // row_tile),
        in_specs=[
            pl.BlockSpec((1, row_tile, 3), lambda bi, ri: (bi, ri, 0)),
            pl.BlockSpec((1, 3, n), lambda bi, ri: (bi, 0, 0)),
        ],
        out_specs=[
            pl.BlockSpec((1, row_tile, 32), lambda bi, ri: (bi, ri, 0)),
            pl.BlockSpec((1, row_tile, 32), lambda bi, ri: (bi, ri, 0)),
        ],
        out_shape=[
            jax.ShapeDtypeStruct((b, n, 32), jnp.float32),
            jax.ShapeDtypeStruct((b, n, 32), jnp.int32),
        ],
    )(point_cloud, pc_t)

    w = outf[..., :k].reshape(b, n * k)
    s_local = outf[..., k]
    target = outi[..., :k].reshape(b, n * k)
    source = jnp.broadcast_to(
        jnp.arange(n, dtype=jnp.int32)[None, :, None], (b, n, k)
    ).reshape(b, n * k)
    edge_index = jnp.stack([source, target], axis=1)
    return edge_index, w, s_local


# speedup vs baseline: 6.7325x; 6.7325x over previous
import functools

import jax
import jax.numpy as jnp
from jax.experimental import pallas as pl

_K = 20
_BETA = 1.0
_LAMBDA = 1.0
_EPS = 1e-06


def _knn_tile_kernel(q_ref, pt_ref, outf_ref, outi_ref, *, n, k):
    q = q_ref[0]
    pt = pt_ref[0]
    r = q.shape[0]

    qsq = jnp.sum(q * q, axis=1, keepdims=True)
    psq = jnp.sum(pt * pt, axis=0, keepdims=True)
    dot = jax.lax.dot(q, pt, preferred_element_type=jnp.float32)
    dist2 = jnp.maximum((qsq + psq) - 2.0 * dot, 0.0)

    iota = jax.lax.broadcasted_iota(jnp.int32, (r, n), 1)
    inf = jnp.float32(jnp.inf)

    vals = dist2
    mins = []
    idxs = []
    for _ in range(k + 1):
        m = jnp.min(vals, axis=1, keepdims=True)
        cand = jnp.where(vals == m, iota, jnp.int32(n))
        idx = jnp.min(cand, axis=1, keepdims=True)
        vals = jnp.where(iota == idx, inf, vals)
        mins.append(m)
        idxs.append(idx)

    md = jnp.concatenate(mins, axis=1)
    mi = jnp.concatenate(idxs, axis=1)

    d = jnp.sqrt(jnp.maximum(md[:, 1:], 1e-12))
    d_i = d[:, k - 1:k]
    sigma = _BETA * d_i + _EPS
    s_local = _LAMBDA * d_i * d_i
    w = jnp.exp(-(d * d) / (2.0 * sigma * sigma))

    padf = jnp.zeros((r, 32 - k - 1), dtype=jnp.float32)
    outf_ref[0] = jnp.concatenate([w, s_local, padf], axis=1)
    padi = jnp.zeros((r, 32 - k), dtype=jnp.int32)
    outi_ref[0] = jnp.concatenate([mi[:, 1:], padi], axis=1)


def kernel(point_cloud):
    b, n, _ = point_cloud.shape
    k = _K
    row_tile = 512
    pc_t = jnp.transpose(point_cloud, (0, 2, 1))

    outf, outi = pl.pallas_call(
        functools.partial(_knn_tile_kernel, n=n, k=k),
        grid=(b, n // row_tile),
        in_specs=[
            pl.BlockSpec((1, row_tile, 3), lambda bi, ri: (bi, ri, 0)),
            pl.BlockSpec((1, 3, n), lambda bi, ri: (bi, 0, 0)),
        ],
        out_specs=[
            pl.BlockSpec((1, row_tile, 32), lambda bi, ri: (bi, ri, 0)),
            pl.BlockSpec((1, row_tile, 32), lambda bi, ri: (bi, ri, 0)),
        ],
        out_shape=[
            jax.ShapeDtypeStruct((b, n, 32), jnp.float32),
            jax.ShapeDtypeStruct((b, n, 32), jnp.int32),
        ],
    )(point_cloud, pc_t)

    w = outf[..., :k].reshape(b, n * k)
    s_local = outf[..., k]
    target = outi[..., :k].reshape(b, n * k)
    source = jnp.broadcast_to(
        jnp.arange(n, dtype=jnp.int32)[None, :, None], (b, n, k)
    ).reshape(b, n * k)
    edge_index = jnp.stack([source, target], axis=1)
    return edge_index, w, s_local


# speedup vs baseline: 7.1150x; 1.0568x over previous
import functools

import jax
import jax.numpy as jnp
from jax.experimental import pallas as pl

_K = 20
_BETA = 1.0
_LAMBDA = 1.0
_EPS = 1e-06


def _knn_tile_kernel(q_ref, pt_ref, outf_ref, outi_ref, *, n, k):
    q = q_ref[0]
    pt = pt_ref[0]
    r = q.shape[0]

    qsq = jnp.sum(q * q, axis=1, keepdims=True)
    psq = jnp.sum(pt * pt, axis=0, keepdims=True)
    dot = jax.lax.dot(q, pt, preferred_element_type=jnp.float32)
    dist2 = jnp.maximum((qsq + psq) - 2.0 * dot, 0.0)

    iota = jax.lax.broadcasted_iota(jnp.int32, (r, n), 1)
    inf = jnp.float32(jnp.inf)

    vals = dist2
    mins = []
    idxs = []
    for _ in range(k + 1):
        m = jnp.min(vals, axis=1, keepdims=True)
        cand = jnp.where(vals == m, iota, jnp.int32(n))
        idx = jnp.min(cand, axis=1, keepdims=True)
        vals = jnp.where(iota == idx, inf, vals)
        mins.append(m)
        idxs.append(idx)

    md = jnp.concatenate(mins, axis=1)
    mi = jnp.concatenate(idxs, axis=1)

    d = jnp.sqrt(jnp.maximum(md[:, 1:], 1e-12))
    d_i = d[:, k - 1:k]
    sigma = _BETA * d_i + _EPS
    s_local = _LAMBDA * d_i * d_i
    w = jnp.exp(-(d * d) / (2.0 * sigma * sigma))

    padf = jnp.zeros((r, 32 - k - 1), dtype=jnp.float32)
    outf_ref[0] = jnp.concatenate([w, s_local, padf], axis=1)
    padi = jnp.zeros((r, 32 - k), dtype=jnp.int32)
    outi_ref[0] = jnp.concatenate([mi[:, 1:], padi], axis=1)


def kernel(point_cloud):
    b, n, _ = point_cloud.shape
    k = _K
    row_tile = 128
    pc_t = jnp.transpose(point_cloud, (0, 2, 1))

    outf, outi = pl.pallas_call(
        functools.partial(_knn_tile_kernel, n=n, k=k),
        grid=(b, n // row_tile),
        in_specs=[
            pl.BlockSpec((1, row_tile, 3), lambda bi, ri: (bi, ri, 0)),
            pl.BlockSpec((1, 3, n), lambda bi, ri: (bi, 0, 0)),
        ],
        out_specs=[
            pl.BlockSpec((1, row_tile, 32), lambda bi, ri: (bi, ri, 0)),
            pl.BlockSpec((1, row_tile, 32), lambda bi, ri: (bi, ri, 0)),
        ],
        out_shape=[
            jax.ShapeDtypeStruct((b, n, 32), jnp.float32),
            jax.ShapeDtypeStruct((b, n, 32), jnp.int32),
        ],
    )(point_cloud, pc_t)

    w = outf[..., :k].reshape(b, n * k)
    s_local = outf[..., k]
    target = outi[..., :k].reshape(b, n * k)
    source = jnp.broadcast_to(
        jnp.arange(n, dtype=jnp.int32)[None, :, None], (b, n, k)
    ).reshape(b, n * k)
    edge_index = jnp.stack([source, target], axis=1)
    return edge_index, w, s_local


# speedup vs baseline: 9.7663x; 1.3726x over previous
import functools

import jax
import jax.numpy as jnp
from jax.experimental import pallas as pl

_K = 20
_BETA = 1.0
_LAMBDA = 1.0
_EPS = 1e-06


def _knn_tile_kernel(q_ref, pt_ref, outf_ref, outi_ref, *, n, k):
    q = q_ref[0]
    pt = pt_ref[0]
    r = q.shape[0]

    qsq = jnp.sum(q * q, axis=1, keepdims=True)
    psq = jnp.sum(pt * pt, axis=0, keepdims=True)
    dot = jax.lax.dot(q, pt, preferred_element_type=jnp.float32)
    dist2 = jnp.maximum((qsq + psq) - 2.0 * dot, 0.0)

    iota = jax.lax.broadcasted_iota(jnp.int32, (r, n), 1).astype(jnp.float32)
    inf = jnp.float32(jnp.inf)
    nf = jnp.float32(n)

    vals = dist2
    mins = []
    idxs = []
    for _ in range(k + 1):
        m = jnp.min(vals, axis=1, keepdims=True)
        cand = jnp.where(vals == m, iota, nf)
        idx = jnp.min(cand, axis=1, keepdims=True)
        vals = jnp.where(iota == idx, inf, vals)
        mins.append(m)
        idxs.append(idx)

    md = jnp.concatenate(mins, axis=1)
    mi = jnp.concatenate(idxs, axis=1).astype(jnp.int32)

    d = jnp.sqrt(jnp.maximum(md[:, 1:], 1e-12))
    d_i = d[:, k - 1:k]
    sigma = _BETA * d_i + _EPS
    s_local = _LAMBDA * d_i * d_i
    w = jnp.exp(-(d * d) / (2.0 * sigma * sigma))

    padf = jnp.zeros((r, 32 - k - 1), dtype=jnp.float32)
    outf_ref[0] = jnp.concatenate([w, s_local, padf], axis=1)
    padi = jnp.zeros((r, 32 - k), dtype=jnp.int32)
    outi_ref[0] = jnp.concatenate([mi[:, 1:], padi], axis=1)


def kernel(point_cloud):
    b, n, _ = point_cloud.shape
    k = _K
    row_tile = 256
    pc_t = jnp.transpose(point_cloud, (0, 2, 1))

    outf, outi = pl.pallas_call(
        functools.partial(_knn_tile_kernel, n=n, k=k),
        grid=(b, n // row_tile),
        in_specs=[
            pl.BlockSpec((1, row_tile, 3), lambda bi, ri: (bi, ri, 0)),
            pl.BlockSpec((1, 3, n), lambda bi, ri: (bi, 0, 0)),
        ],
        out_specs=[
            pl.BlockSpec((1, row_tile, 32), lambda bi, ri: (bi, ri, 0)),
            pl.BlockSpec((1, row_tile, 32), lambda bi, ri: (bi, ri, 0)),
        ],
        out_shape=[
            jax.ShapeDtypeStruct((b, n, 32), jnp.float32),
            jax.ShapeDtypeStruct((b, n, 32), jnp.int32),
        ],
    )(point_cloud, pc_t)

    w = outf[..., :k].reshape(b, n * k)
    s_local = outf[..., k]
    target = outi[..., :k].reshape(b, n * k)
    source = jnp.broadcast_to(
        jnp.arange(n, dtype=jnp.int32)[None, :, None], (b, n, k)
    ).reshape(b, n * k)
    edge_index = jnp.stack([source, target], axis=1)
    return edge_index, w, s_local
